# Initial kernel scaffold; baseline (speedup 1.0000x reference)
import jax
import jax.numpy as jnp
from jax.experimental import pallas as pl


def kernel(x, edge_index, edge_attr, We1, be1, We2, be2, Wc1, bc1, Wc2, bc2, Wn1, bn1, Wn2, bn2):
    raise NotImplementedError("write your pallas kernel here")


# speedup vs baseline: 2.1124x; 2.1124x over previous
import functools

import jax
import jax.numpy as jnp
from jax import lax
from jax.experimental import pallas as pl
from jax.experimental.pallas import tpu as pltpu
from jax.experimental.pallas import tpu_sc as plsc

NC, NS = 2, 16
NW = NC * NS
F32 = jnp.float32


def _silu(v):
    return v * jax.nn.sigmoid(v)


def _make_gather(N, E, fdim, cdim, tdim, chunk):
    epw = E // NW
    nch = epw // chunk
    mesh = plsc.VectorSubcoreMesh(core_axis_name="c", subcore_axis_name="s", num_cores=NC, num_subcores=NS)

    @functools.partial(
        pl.kernel,
        out_type=(
            jax.ShapeDtypeStruct((E, tdim), F32),
            jax.ShapeDtypeStruct((E, tdim), F32),
        ),
        mesh=mesh,
        scratch_types=[
            pltpu.VMEM((chunk,), jnp.int32),
            pltpu.VMEM((chunk,), jnp.int32),
            pltpu.VMEM((chunk, tdim), F32),
            pltpu.VMEM((chunk, tdim), F32),
            pltpu.SemaphoreType.DMA,
        ],
    )
    def gather_k(tab, idx0_h, idx1_h, g0_o, g1_o,
                 idx0_v, idx1_v, g0_v, g1_v, sem):
        wid = lax.axis_index("s") * NC + lax.axis_index("c")
        base = wid * epw

        def body(j, carry):
            off = base + j * chunk
            pltpu.sync_copy(idx0_h.at[pl.ds(off, chunk)], idx0_v)
            pltpu.sync_copy(idx1_h.at[pl.ds(off, chunk)], idx1_v)
            cp0 = pltpu.async_copy(tab.at[idx0_v], g0_v, sem)
            cp1 = pltpu.async_copy(tab.at[idx1_v], g1_v, sem)
            cp0.wait(); cp1.wait()
            pltpu.sync_copy(g0_v, g0_o.at[pl.ds(off, chunk)])
            pltpu.sync_copy(g1_v, g1_o.at[pl.ds(off, chunk)])
            return carry

        lax.fori_loop(0, nch, body, 0)

    return gather_k


def _make_scatter(N, E, vdim, chunk):
    epw = E // NW
    nch = epw // chunk
    mesh = plsc.VectorSubcoreMesh(core_axis_name="c", subcore_axis_name="s", num_cores=NC, num_subcores=NS)

    @functools.partial(
        pl.kernel,
        out_type=jax.ShapeDtypeStruct((NC, N, vdim), F32),
        mesh=mesh,
        scratch_types=[
            pltpu.VMEM((chunk,), jnp.int32),
            pltpu.VMEM((chunk, vdim), F32),
            pltpu.VMEM_SHARED((N, vdim), F32),
            pltpu.SemaphoreType.DMA,
        ],
    )
    def scatter_k(idx1_h, v_h, z_h, p_o, idx_v, v_v, acc, sem):
        cid = lax.axis_index("c")
        sid = lax.axis_index("s")
        wid = cid * NS + sid
        base = wid * epw

        @pl.when(sid == 0)
        def _init():
            pltpu.sync_copy(z_h, acc)

        plsc.subcore_barrier()

        def body(j, carry):
            off = base + j * chunk
            pltpu.sync_copy(idx1_h.at[pl.ds(off, chunk)], idx_v)
            pltpu.sync_copy(v_h.at[pl.ds(off, chunk)], v_v)
            pltpu.sync_copy(v_v, acc.at[idx_v], add=True)
            return carry

        lax.fori_loop(0, nch, body, 0)
        plsc.subcore_barrier()

        @pl.when(sid == 0)
        def _drain():
            pltpu.sync_copy(acc, p_o.at[cid])

    return scatter_k


def _edge_body(fdim, cdim, g0_r, g1_r, ea_r, wa_r, wb_r, wea_r, wd_r, be1_r,
               we2_r, be2_r, wc1_r, bc1_r, wc2_r, bc2_r, v_r):
    g0 = g0_r[...]
    g1 = g1_r[...]
    f0 = g0[:, :fdim].astype(jnp.bfloat16)
    f1 = g1[:, :fdim].astype(jnp.bfloat16)
    rel = g0[:, fdim:fdim + cdim] - g1[:, fdim:fdim + cdim]
    rd = jnp.sum(rel * rel, axis=1, keepdims=True)
    base = (jnp.dot(ea_r[...], wea_r[...], preferred_element_type=F32)
            + rd * wd_r[...] + be1_r[...])
    wa = wa_r[...]
    wb = wb_r[...]
    p0a = jnp.dot(f0, wa, preferred_element_type=F32)
    p0b = jnp.dot(f0, wb, preferred_element_type=F32)
    p1a = jnp.dot(f1, wa, preferred_element_type=F32)
    p1b = jnp.dot(f1, wb, preferred_element_type=F32)
    h1 = _silu(p0a + p1b + base)
    h2 = _silu(p1a + p0b + base)
    we2 = we2_r[...]
    m1 = _silu(jnp.dot(h1, we2, preferred_element_type=F32) + be2_r[...])
    m2 = _silu(jnp.dot(h2, we2, preferred_element_type=F32) + be2_r[...])
    t = _silu(jnp.dot(m1, wc1_r[...], preferred_element_type=F32) + bc1_r[...])
    wij = jnp.sum(t * wc2_r[...], axis=1, keepdims=True) + bc2_r[...]
    eb = m2.shape[0]
    v_r[...] = jnp.concatenate(
        [m2, wij * rel, jnp.zeros((eb, v_r.shape[1] - m2.shape[1] - rel.shape[1]), F32)],
        axis=1)


def _run_edge_mlp(G0, G1, EA, wa, wb, wea, wd, be1, we2, be2,
                  wc1, bc1, wc2r, bc2, fdim, cdim, eb):
    E, tdim = G0.shape
    mdim = we2.shape[1]
    grid = (E // eb,)

    def em(i):
        return (i, 0)

    def w0(i):
        return (0, 0)

    return pl.pallas_call(
        functools.partial(_edge_body, fdim, cdim),
        grid=grid,
        in_specs=[
            pl.BlockSpec((eb, tdim), em),
            pl.BlockSpec((eb, tdim), em),
            pl.BlockSpec((eb, EA.shape[1]), em),
            pl.BlockSpec(wa.shape, w0),
            pl.BlockSpec(wb.shape, w0),
            pl.BlockSpec(wea.shape, w0),
            pl.BlockSpec(wd.shape, w0),
            pl.BlockSpec(be1.shape, w0),
            pl.BlockSpec(we2.shape, w0),
            pl.BlockSpec(be2.shape, w0),
            pl.BlockSpec(wc1.shape, w0),
            pl.BlockSpec(bc1.shape, w0),
            pl.BlockSpec(wc2r.shape, w0),
            pl.BlockSpec(bc2.shape, w0),
        ],
        out_specs=pl.BlockSpec((eb, 128), em),
        out_shape=jax.ShapeDtypeStruct((E, 128), F32),
        compiler_params=pltpu.CompilerParams(
            dimension_semantics=("arbitrary",)),
    )(G0, G1, EA, wa, wb, wea, wd, be1, we2, be2, wc1, bc1, wc2r, bc2)


def _node_body(mdim, cdim, f_r, c_r, sp0_r, sp1_r, wn1f_r, wn1m_r, bn1_r,
               wn2_r, bn2_r, co_r, hid_r):
    f = f_r[...]
    s = sp0_r[...] + sp1_r[...]
    m = s[:, :mdim]
    pre = (jnp.dot(f, wn1f_r[...], preferred_element_type=F32)
           + jnp.dot(m, wn1m_r[...], preferred_element_type=F32)
           + bn1_r[...])
    h = jnp.dot(_silu(pre), wn2_r[...], preferred_element_type=F32) + bn2_r[...]
    hid_r[...] = f + h
    co_r[...] = c_r[...] + s[:, mdim:mdim + cdim]


def _run_node_mlp(feats, ctab, sp0, sp1, wn1f, wn1m, bn1, wn2, bn2, mdim, nb):
    N, fdim = feats.shape
    cdim = ctab.shape[1]
    vdim = sp0.shape[1]
    grid = (N // nb,)

    def em(i):
        return (i, 0)

    def w0(i):
        return (0, 0)

    return pl.pallas_call(
        functools.partial(_node_body, mdim, cdim),
        grid=grid,
        in_specs=[
            pl.BlockSpec((nb, fdim), em),
            pl.BlockSpec((nb, cdim), em),
            pl.BlockSpec((nb, vdim), em),
            pl.BlockSpec((nb, vdim), em),
            pl.BlockSpec(wn1f.shape, w0),
            pl.BlockSpec(wn1m.shape, w0),
            pl.BlockSpec(bn1.shape, w0),
            pl.BlockSpec(wn2.shape, w0),
            pl.BlockSpec(bn2.shape, w0),
        ],
        out_specs=[
            pl.BlockSpec((nb, cdim), em),
            pl.BlockSpec((nb, fdim), em),
        ],
        out_shape=[
            jax.ShapeDtypeStruct((N, cdim), F32),
            jax.ShapeDtypeStruct((N, fdim), F32),
        ],
        compiler_params=pltpu.CompilerParams(
            dimension_semantics=("arbitrary",)),
    )(feats, ctab, sp0, sp1, wn1f, wn1m, bn1, wn2, bn2)


def kernel(x, edge_index, edge_attr, We1, be1, We2, be2, Wc1, bc1, Wc2, bc2,
           Wn1, bn1, Wn2, bn2):
    pos_dim = 3
    N = x.shape[0]
    E = edge_index.shape[1]
    fdim = x.shape[1] - pos_dim
    cdim = 16
    adim = edge_attr.shape[1]
    hid = We1.shape[1]
    hp = 640
    mdim = We2.shape[1]

    coors = x[:, :pos_dim]
    feats = x[:, pos_dim:]
    ctab = jnp.pad(coors, ((0, 0), (0, cdim - pos_dim)))
    tdim = 256
    tab = jnp.pad(jnp.concatenate([feats, ctab], axis=1),
                  ((0, 0), (0, tdim - fdim - cdim)))
    idx0 = edge_index[0].astype(jnp.int32)
    idx1 = edge_index[1].astype(jnp.int32)

    pad = ((0, 0), (0, hp - hid))
    wa = jnp.pad(We1[:fdim], pad).astype(jnp.bfloat16)
    wb = jnp.pad(We1[fdim:2 * fdim], pad).astype(jnp.bfloat16)
    wea = jnp.pad(We1[2 * fdim:2 * fdim + adim], pad)
    wd = jnp.pad(We1[2 * fdim + adim], (0, hp - hid)).reshape(1, hp)
    be1p = jnp.pad(be1, (0, hp - hid)).reshape(1, hp)
    we2p = jnp.pad(We2, ((0, hp - hid), (0, 0)))
    wc2r = Wc2.reshape(1, -1)
    bc2r = bc2.reshape(1, 1)
    be2r = be2.reshape(1, -1)
    bc1r = bc1.reshape(1, -1)
    wn1f = Wn1[:fdim]
    wn1m = Wn1[fdim:]
    bn1r = bn1.reshape(1, -1)
    bn2r = bn2.reshape(1, -1)

    chunk = 80
    G0, G1 = _make_gather(N, E, fdim, cdim, tdim, chunk)(tab, idx0, idx1)

    V = _run_edge_mlp(G0, G1, edge_attr, wa, wb, wea, wd, be1p,
                      we2p, be2r, Wc1, bc1r, wc2r, bc2r,
                      fdim=fdim, cdim=cdim, eb=1280)

    z = jnp.zeros((N, 128), F32)
    Sp = _make_scatter(N, E, 128, chunk)(idx1, V, z)

    co, hidden = _run_node_mlp(feats, ctab, Sp[0], Sp[1],
                               wn1f, wn1m, bn1r, Wn2, bn2r, mdim=mdim,
                               nb=2000)

    return jnp.concatenate([co[:, :pos_dim], hidden], axis=-1)


# speedup vs baseline: 2.3160x; 1.0964x over previous
import functools

import jax
import jax.numpy as jnp
from jax import lax
from jax.experimental import pallas as pl
from jax.experimental.pallas import tpu as pltpu
from jax.experimental.pallas import tpu_sc as plsc

NC, NS = 2, 16
NW = NC * NS
F32 = jnp.float32
BF16 = jnp.bfloat16


def _silu(v):
    return v * jax.nn.sigmoid(v)


def _make_gather(N, E, tdim, chunk):
    epw = E // NW
    nch = epw // chunk
    mesh = plsc.VectorSubcoreMesh(core_axis_name="c", subcore_axis_name="s", num_cores=NC, num_subcores=NS)

    @functools.partial(
        pl.kernel,
        out_type=(
            jax.ShapeDtypeStruct((E, tdim), jnp.int32),
            jax.ShapeDtypeStruct((E, tdim), jnp.int32),
        ),
        mesh=mesh,
        scratch_types=[
            pltpu.VMEM((chunk,), jnp.int32),
            pltpu.VMEM((chunk,), jnp.int32),
            pltpu.VMEM((chunk, tdim), jnp.int32),
            pltpu.VMEM((chunk, tdim), jnp.int32),
            pltpu.SemaphoreType.DMA,
        ],
    )
    def gather_k(tab, idx0_h, idx1_h, g0_o, g1_o,
                 idx0_v, idx1_v, g0_v, g1_v, sem):
        wid = lax.axis_index("s") * NC + lax.axis_index("c")
        base = wid * epw

        def body(j, carry):
            off = base + j * chunk
            pltpu.sync_copy(idx0_h.at[pl.ds(off, chunk)], idx0_v)
            pltpu.sync_copy(idx1_h.at[pl.ds(off, chunk)], idx1_v)
            cp0 = pltpu.async_copy(tab.at[idx0_v], g0_v, sem)
            cp1 = pltpu.async_copy(tab.at[idx1_v], g1_v, sem)
            cp0.wait(); cp1.wait()
            pltpu.sync_copy(g0_v, g0_o.at[pl.ds(off, chunk)])
            pltpu.sync_copy(g1_v, g1_o.at[pl.ds(off, chunk)])
            return carry

        lax.fori_loop(0, nch, body, 0)

    return gather_k


def _make_scatter(N, E, vdim, chunk):
    epw = E // NW
    nch = epw // chunk
    mesh = plsc.VectorSubcoreMesh(core_axis_name="c", subcore_axis_name="s", num_cores=NC, num_subcores=NS)

    @functools.partial(
        pl.kernel,
        out_type=jax.ShapeDtypeStruct((NC, N, vdim), F32),
        mesh=mesh,
        scratch_types=[
            pltpu.VMEM((chunk,), jnp.int32),
            pltpu.VMEM((chunk, vdim), F32),
            pltpu.VMEM_SHARED((N, vdim), F32),
            pltpu.SemaphoreType.DMA,
        ],
    )
    def scatter_k(idx1_h, v_h, z_h, p_o, idx_v, v_v, acc, sem):
        cid = lax.axis_index("c")
        sid = lax.axis_index("s")
        wid = cid * NS + sid
        base = wid * epw

        @pl.when(sid == 0)
        def _init():
            pltpu.sync_copy(z_h, acc)

        plsc.subcore_barrier()

        def body(j, carry):
            off = base + j * chunk
            pltpu.sync_copy(idx1_h.at[pl.ds(off, chunk)], idx_v)
            pltpu.sync_copy(v_h.at[pl.ds(off, chunk)], v_v)
            pltpu.sync_copy(v_v, acc.at[idx_v], add=True)
            return carry

        lax.fori_loop(0, nch, body, 0)
        plsc.subcore_barrier()

        @pl.when(sid == 0)
        def _drain():
            pltpu.sync_copy(acc, p_o.at[cid])

    return scatter_k


def _unpack_feats(g):
    lo = lax.bitcast_convert_type(g << 16, F32)
    hi = lax.bitcast_convert_type(g & jnp.int32(-65536), F32)
    return jnp.concatenate([lo, hi], axis=1).astype(BF16)


def _edge_body(pdim, cdim, g0_r, g1_r, ea_r, wa_r, wb_r, wea_r,
               wd_r, be1_r, we2_r, be2_r, wc1_r, bc1_r, wc2_r, bc2_r, v_r):
    g0 = g0_r[...]
    g1 = g1_r[...]
    f0 = _unpack_feats(g0[:, :pdim])
    f1 = _unpack_feats(g1[:, :pdim])
    c0 = lax.bitcast_convert_type(g0[:, pdim:pdim + cdim], F32)
    c1 = lax.bitcast_convert_type(g1[:, pdim:pdim + cdim], F32)
    rel = c0 - c1
    rd = jnp.sum(rel * rel, axis=1, keepdims=True)
    base = (jnp.dot(ea_r[...], wea_r[...], preferred_element_type=F32)
            + rd * wd_r[...] + be1_r[...])
    wa = wa_r[...]
    wb = wb_r[...]
    p0a = jnp.dot(f0, wa, preferred_element_type=F32)
    p0b = jnp.dot(f0, wb, preferred_element_type=F32)
    p1a = jnp.dot(f1, wa, preferred_element_type=F32)
    p1b = jnp.dot(f1, wb, preferred_element_type=F32)
    h1 = _silu(p0a + p1b + base)
    h2 = _silu(p1a + p0b + base)
    we2 = we2_r[...]
    m1 = _silu(jnp.dot(h1, we2, preferred_element_type=F32) + be2_r[...])
    m2 = _silu(jnp.dot(h2, we2, preferred_element_type=F32) + be2_r[...])
    t = _silu(jnp.dot(m1, wc1_r[...], preferred_element_type=F32) + bc1_r[...])
    wij = jnp.sum(t * wc2_r[...], axis=1, keepdims=True) + bc2_r[...]
    eb = m2.shape[0]
    zpad = v_r.shape[1] - m2.shape[1] - rel.shape[1]
    v_r[...] = jnp.concatenate(
        [m2, wij * rel, jnp.zeros((eb, zpad), F32)], axis=1)


def _run_edge_mlp(G0, G1, EA, wa, wb, wea, wd, be1, we2, be2,
                  wc1, bc1, wc2r, bc2, pdim, cdim, eb):
    E, tdim = G0.shape
    mdim = we2.shape[1]
    vdim = 128
    grid = (E // eb,)

    def em(i):
        return (i, 0)

    def w0(i):
        return (0, 0)

    return pl.pallas_call(
        functools.partial(_edge_body, pdim, cdim),
        grid=grid,
        in_specs=[
            pl.BlockSpec((eb, tdim), em),
            pl.BlockSpec((eb, tdim), em),
            pl.BlockSpec((eb, EA.shape[1]), em),
            pl.BlockSpec(wa.shape, w0),
            pl.BlockSpec(wb.shape, w0),
            pl.BlockSpec(wea.shape, w0),
            pl.BlockSpec(wd.shape, w0),
            pl.BlockSpec(be1.shape, w0),
            pl.BlockSpec(we2.shape, w0),
            pl.BlockSpec(be2.shape, w0),
            pl.BlockSpec(wc1.shape, w0),
            pl.BlockSpec(bc1.shape, w0),
            pl.BlockSpec(wc2r.shape, w0),
            pl.BlockSpec(bc2.shape, w0),
        ],
        out_specs=pl.BlockSpec((eb, vdim), em),
        out_shape=jax.ShapeDtypeStruct((E, vdim), F32),
        compiler_params=pltpu.CompilerParams(
            dimension_semantics=("arbitrary",)),
    )(G0, G1, EA, wa, wb, wea, wd, be1, we2, be2, wc1, bc1, wc2r, bc2)


def _node_body(mdim, cdim, f_r, c_r, sp0_r, sp1_r, wn1f_r, wn1m_r, bn1_r,
               wn2_r, bn2_r, co_r, hid_r):
    f = f_r[...]
    s = sp0_r[...] + sp1_r[...]
    m = s[:, :mdim]
    pre = (jnp.dot(f, wn1f_r[...], preferred_element_type=F32)
           + jnp.dot(m, wn1m_r[...], preferred_element_type=F32)
           + bn1_r[...])
    h = jnp.dot(_silu(pre), wn2_r[...], preferred_element_type=F32) + bn2_r[...]
    hid_r[...] = f + h
    co_r[...] = c_r[...] + s[:, mdim:mdim + cdim]


def _run_node_mlp(feats, ctab, sp0, sp1, wn1f, wn1m, bn1, wn2, bn2, mdim, nb):
    N, fdim = feats.shape
    cdim = ctab.shape[1]
    vdim = sp0.shape[1]
    grid = (N // nb,)

    def em(i):
        return (i, 0)

    def w0(i):
        return (0, 0)

    return pl.pallas_call(
        functools.partial(_node_body, mdim, cdim),
        grid=grid,
        in_specs=[
            pl.BlockSpec((nb, fdim), em),
            pl.BlockSpec((nb, cdim), em),
            pl.BlockSpec((nb, vdim), em),
            pl.BlockSpec((nb, vdim), em),
            pl.BlockSpec(wn1f.shape, w0),
            pl.BlockSpec(wn1m.shape, w0),
            pl.BlockSpec(bn1.shape, w0),
            pl.BlockSpec(wn2.shape, w0),
            pl.BlockSpec(bn2.shape, w0),
        ],
        out_specs=[
            pl.BlockSpec((nb, cdim), em),
            pl.BlockSpec((nb, fdim), em),
        ],
        out_shape=[
            jax.ShapeDtypeStruct((N, cdim), F32),
            jax.ShapeDtypeStruct((N, fdim), F32),
        ],
        compiler_params=pltpu.CompilerParams(
            dimension_semantics=("arbitrary",)),
    )(feats, ctab, sp0, sp1, wn1f, wn1m, bn1, wn2, bn2)


def kernel(x, edge_index, edge_attr, We1, be1, We2, be2, Wc1, bc1, Wc2, bc2,
           Wn1, bn1, Wn2, bn2):
    pos_dim = 3
    N = x.shape[0]
    E = edge_index.shape[1]
    fdim = x.shape[1] - pos_dim
    pdim = fdim // 2
    cdim = 16
    adim = edge_attr.shape[1]
    hid = We1.shape[1]
    hp = 640
    mdim = We2.shape[1]

    coors = x[:, :pos_dim]
    feats = x[:, pos_dim:]
    ctab = jnp.pad(coors, ((0, 0), (0, cdim - pos_dim)))
    fpack = lax.bitcast_convert_type(
        feats.astype(BF16).reshape(N, pdim, 2), jnp.int32)
    tdim = 128
    tab = jnp.concatenate(
        [fpack, lax.bitcast_convert_type(ctab, jnp.int32),
         jnp.zeros((N, tdim - pdim - cdim), jnp.int32)], axis=1)
    idx0 = edge_index[0].astype(jnp.int32)
    idx1 = edge_index[1].astype(jnp.int32)

    perm = jnp.concatenate([jnp.arange(0, fdim, 2), jnp.arange(1, fdim, 2)])
    pad = ((0, 0), (0, hp - hid))
    wa = jnp.pad(We1[:fdim], pad).astype(BF16)[perm]
    wb = jnp.pad(We1[fdim:2 * fdim], pad).astype(BF16)[perm]
    wea = jnp.pad(We1[2 * fdim:2 * fdim + adim], pad)
    wd = jnp.pad(We1[2 * fdim + adim], (0, hp - hid)).reshape(1, hp)
    be1p = jnp.pad(be1, (0, hp - hid)).reshape(1, hp)
    we2p = jnp.pad(We2, ((0, hp - hid), (0, 0)))
    wc2r = Wc2.reshape(1, -1)
    bc2r = bc2.reshape(1, 1)
    be2r = be2.reshape(1, -1)
    bc1r = bc1.reshape(1, -1)
    wn1f = Wn1[:fdim]
    wn1m = Wn1[fdim:]
    bn1r = bn1.reshape(1, -1)
    bn2r = bn2.reshape(1, -1)

    chunk = 80
    G0, G1 = _make_gather(N, E, tdim, chunk)(tab, idx0, idx1)

    V = _run_edge_mlp(G0, G1, edge_attr, wa, wb, wea, wd, be1p,
                      we2p, be2r, Wc1, bc1r, wc2r, bc2r, pdim=pdim,
                      cdim=cdim, eb=1280)

    vdim = 128
    z = jnp.zeros((N, vdim), F32)
    Sp = _make_scatter(N, E, vdim, chunk)(idx1, V, z)

    co, hidden = _run_node_mlp(feats, ctab, Sp[0], Sp[1],
                               wn1f, wn1m, bn1r, Wn2, bn2r, mdim=mdim,
                               nb=2000)

    return jnp.concatenate([co[:, :pos_dim], hidden], axis=-1)


# speedup vs baseline: 3.0714x; 1.3262x over previous
import functools

import jax
import jax.numpy as jnp
from jax import lax
from jax.experimental import pallas as pl
from jax.experimental.pallas import tpu as pltpu
from jax.experimental.pallas import tpu_sc as plsc

NC, NS = 2, 16
NW = NC * NS
F32 = jnp.float32
BF16 = jnp.bfloat16


def _silu(v):
    return v * jax.nn.sigmoid(v)


def _make_gather(N, E, tdim, chunk):
    epw = E // NW
    nch = epw // chunk
    mesh = plsc.VectorSubcoreMesh(core_axis_name="c", subcore_axis_name="s", num_cores=NC, num_subcores=NS)

    @functools.partial(
        pl.kernel,
        out_type=(
            jax.ShapeDtypeStruct((E, tdim), jnp.int32),
            jax.ShapeDtypeStruct((E, tdim), jnp.int32),
        ),
        mesh=mesh,
        scratch_types=[
            pltpu.VMEM((chunk,), jnp.int32),
            pltpu.VMEM((chunk,), jnp.int32),
            pltpu.VMEM((chunk, tdim), jnp.int32),
            pltpu.VMEM((chunk, tdim), jnp.int32),
            pltpu.SemaphoreType.DMA,
        ],
    )
    def gather_k(tab, idx0_h, idx1_h, g0_o, g1_o,
                 idx0_v, idx1_v, g0_v, g1_v, sem):
        wid = lax.axis_index("s") * NC + lax.axis_index("c")
        base = wid * epw

        def body(j, carry):
            off = base + j * chunk
            pltpu.sync_copy(idx0_h.at[pl.ds(off, chunk)], idx0_v)
            pltpu.sync_copy(idx1_h.at[pl.ds(off, chunk)], idx1_v)
            cp0 = pltpu.async_copy(tab.at[idx0_v], g0_v, sem)
            cp1 = pltpu.async_copy(tab.at[idx1_v], g1_v, sem)
            cp0.wait(); cp1.wait()
            pltpu.sync_copy(g0_v, g0_o.at[pl.ds(off, chunk)])
            pltpu.sync_copy(g1_v, g1_o.at[pl.ds(off, chunk)])
            return carry

        lax.fori_loop(0, nch, body, 0)

    return gather_k


def _make_scatter(N, E, vdim, chunk):
    epw = E // NW
    nch = epw // chunk
    mesh = plsc.VectorSubcoreMesh(core_axis_name="c", subcore_axis_name="s", num_cores=NC, num_subcores=NS)

    @functools.partial(
        pl.kernel,
        out_type=jax.ShapeDtypeStruct((NC, N, vdim), F32),
        mesh=mesh,
        scratch_types=[
            pltpu.VMEM((chunk,), jnp.int32),
            pltpu.VMEM((chunk, vdim), F32),
            pltpu.VMEM_SHARED((N, vdim), F32),
            pltpu.SemaphoreType.DMA,
        ],
    )
    def scatter_k(idx1_h, v_h, z_h, p_o, idx_v, v_v, acc, sem):
        cid = lax.axis_index("c")
        sid = lax.axis_index("s")
        wid = cid * NS + sid
        base = wid * epw

        @pl.when(sid == 0)
        def _init():
            pltpu.sync_copy(z_h, acc)

        plsc.subcore_barrier()

        def body(j, carry):
            off = base + j * chunk
            pltpu.sync_copy(idx1_h.at[pl.ds(off, chunk)], idx_v)
            pltpu.sync_copy(v_h.at[pl.ds(off, chunk)], v_v)
            pltpu.sync_copy(v_v, acc.at[idx_v], add=True)
            return carry

        lax.fori_loop(0, nch, body, 0)
        plsc.subcore_barrier()

        @pl.when(sid == 0)
        def _drain():
            pltpu.sync_copy(acc, p_o.at[cid])

    return scatter_k


def _unpack_feats(g):
    lo = lax.bitcast_convert_type(g << 16, F32)
    hi = lax.bitcast_convert_type(g & jnp.int32(-65536), F32)
    return jnp.concatenate([lo, hi], axis=1).astype(BF16)


def _edge_body(pdim, cdim, g0_r, g1_r, ea_r, wa_r, wb_r, wea_r, wd_r, be1_r,
               we2_r, be2_r, wc1_r, bc1_r, wc2_r, bc2_r, v_r):
    g0 = g0_r[...]
    g1 = g1_r[...]
    f0 = _unpack_feats(g0[:, :pdim])
    f1 = _unpack_feats(g1[:, :pdim])
    c0 = lax.bitcast_convert_type(g0[:, pdim:pdim + cdim], F32)
    c1 = lax.bitcast_convert_type(g1[:, pdim:pdim + cdim], F32)
    rel = c0 - c1
    rd = jnp.sum(rel * rel, axis=1, keepdims=True)
    ea = ea_r[...]
    eb = ea.shape[0]
    base = (jnp.dot(ea, wea_r[...], preferred_element_type=F32)
            + rd * wd_r[...] + be1_r[...])
    wa = wa_r[...]
    wb = wb_r[...]
    p0a = jnp.dot(f0, wa, preferred_element_type=F32)
    p0b = jnp.dot(f0, wb, preferred_element_type=F32)
    p1a = jnp.dot(f1, wa, preferred_element_type=F32)
    p1b = jnp.dot(f1, wb, preferred_element_type=F32)
    h1 = _silu(p0a + p1b + base)
    h2 = _silu(p1a + p0b + base)
    we2 = we2_r[...]
    m1 = _silu(jnp.dot(h1, we2, preferred_element_type=F32) + be2_r[...])
    m2 = _silu(jnp.dot(h2, we2, preferred_element_type=F32) + be2_r[...])
    t = _silu(jnp.dot(m1, wc1_r[...], preferred_element_type=F32) + bc1_r[...])
    wij = jnp.sum(t * wc2_r[...], axis=1, keepdims=True) + bc2_r[...]
    zpad = v_r.shape[1] - m2.shape[1] - rel.shape[1]
    v_r[...] = jnp.concatenate(
        [m2, wij * rel, jnp.zeros((eb, zpad), F32)], axis=1)


def _run_edge_mlp(G0, G1, EA, wa, wb, wea, wd, be1, we2, be2,
                  wc1, bc1, wc2r, bc2, pdim, cdim, eb):
    E, tdim = G0.shape
    mdim = we2.shape[1]
    vdim = 128
    grid = (E // eb,)

    def em(i):
        return (i, 0)

    def w0(i):
        return (0, 0)

    return pl.pallas_call(
        functools.partial(_edge_body, pdim, cdim),
        grid=grid,
        in_specs=[
            pl.BlockSpec((eb, tdim), em),
            pl.BlockSpec((eb, tdim), em),
            pl.BlockSpec((eb, EA.shape[1]), em),
            pl.BlockSpec(wa.shape, w0),
            pl.BlockSpec(wb.shape, w0),
            pl.BlockSpec(wea.shape, w0),
            pl.BlockSpec(wd.shape, w0),
            pl.BlockSpec(be1.shape, w0),
            pl.BlockSpec(we2.shape, w0),
            pl.BlockSpec(be2.shape, w0),
            pl.BlockSpec(wc1.shape, w0),
            pl.BlockSpec(bc1.shape, w0),
            pl.BlockSpec(wc2r.shape, w0),
            pl.BlockSpec(bc2.shape, w0),
        ],
        out_specs=pl.BlockSpec((eb, vdim), em),
        out_shape=jax.ShapeDtypeStruct((E, vdim), F32),
        compiler_params=pltpu.CompilerParams(
            dimension_semantics=("arbitrary",)),
    )(G0, G1, EA, wa, wb, wea, wd, be1, we2, be2, wc1, bc1, wc2r, bc2)


def _node_body(mdim, cdim, nsp, f_r, c_r, *rest):
    sp_rs = rest[:nsp]
    wn1f_r, wn1m_r, bn1_r, wn2_r, bn2_r, co_r, hid_r = rest[nsp:]
    f = f_r[...]
    s = sp_rs[0][...]
    for r in sp_rs[1:]:
        s = s + r[...]
    m = s[:, :mdim]
    pre = (jnp.dot(f, wn1f_r[...], preferred_element_type=F32)
           + jnp.dot(m, wn1m_r[...], preferred_element_type=F32)
           + bn1_r[...])
    h = jnp.dot(_silu(pre), wn2_r[...], preferred_element_type=F32) + bn2_r[...]
    hid_r[...] = f + h
    co_r[...] = c_r[...] + s[:, mdim:mdim + cdim]


def _run_node_mlp(feats, ctab, sps, wn1f, wn1m, bn1, wn2, bn2, mdim, nb):
    N, fdim = feats.shape
    cdim = ctab.shape[1]
    vdim = sps[0].shape[1]
    grid = (N // nb,)

    def em(i):
        return (i, 0)

    def w0(i):
        return (0, 0)

    return pl.pallas_call(
        functools.partial(_node_body, mdim, cdim, len(sps)),
        grid=grid,
        in_specs=[
            pl.BlockSpec((nb, fdim), em),
            pl.BlockSpec((nb, cdim), em),
        ] + [pl.BlockSpec((nb, vdim), em) for _ in sps] + [
            pl.BlockSpec(wn1f.shape, w0),
            pl.BlockSpec(wn1m.shape, w0),
            pl.BlockSpec(bn1.shape, w0),
            pl.BlockSpec(wn2.shape, w0),
            pl.BlockSpec(bn2.shape, w0),
        ],
        out_specs=[
            pl.BlockSpec((nb, cdim), em),
            pl.BlockSpec((nb, fdim), em),
        ],
        out_shape=[
            jax.ShapeDtypeStruct((N, cdim), F32),
            jax.ShapeDtypeStruct((N, fdim), F32),
        ],
        compiler_params=pltpu.CompilerParams(
            dimension_semantics=("arbitrary",)),
    )(feats, ctab, *sps, wn1f, wn1m, bn1, wn2, bn2)


def kernel(x, edge_index, edge_attr, We1, be1, We2, be2, Wc1, bc1, Wc2, bc2,
           Wn1, bn1, Wn2, bn2):
    pos_dim = 3
    N = x.shape[0]
    E = edge_index.shape[1]
    fdim = x.shape[1] - pos_dim
    pdim = fdim // 2
    cdim = 16
    adim = edge_attr.shape[1]
    hid = We1.shape[1]
    hp = 640
    mdim = We2.shape[1]

    coors = x[:, :pos_dim]
    feats = x[:, pos_dim:]
    ctab = jnp.pad(coors, ((0, 0), (0, cdim - pos_dim)))
    fpack = lax.bitcast_convert_type(
        feats.astype(BF16).reshape(N, pdim, 2), jnp.int32)
    tdim = 128
    tab = jnp.concatenate(
        [fpack, lax.bitcast_convert_type(ctab, jnp.int32),
         jnp.zeros((N, tdim - pdim - cdim), jnp.int32)], axis=1)
    idx0 = edge_index[0].astype(jnp.int32)
    idx1 = edge_index[1].astype(jnp.int32)

    perm = jnp.concatenate([jnp.arange(0, fdim, 2), jnp.arange(1, fdim, 2)])
    pad = ((0, 0), (0, hp - hid))
    wa = jnp.pad(We1[:fdim], pad).astype(BF16)[perm]
    wb = jnp.pad(We1[fdim:2 * fdim], pad).astype(BF16)[perm]
    wea = jnp.pad(We1[2 * fdim:2 * fdim + adim], pad)
    wd = jnp.pad(We1[2 * fdim + adim], (0, hp - hid)).reshape(1, hp)
    be1p = jnp.pad(be1, (0, hp - hid)).reshape(1, hp)
    we2p = jnp.pad(We2, ((0, hp - hid), (0, 0)))
    wc2r = Wc2.reshape(1, -1)
    bc2r = bc2.reshape(1, 1)
    be2r = be2.reshape(1, -1)
    bc1r = bc1.reshape(1, -1)
    wn1f = Wn1[:fdim]
    wn1m = Wn1[fdim:]
    bn1r = bn1.reshape(1, -1)
    bn2r = bn2.reshape(1, -1)

    K = 2
    Ec = E // K
    chunk = 200
    vdim = 128
    z = jnp.zeros((N, vdim), F32)
    gather = _make_gather(N, Ec, tdim, chunk)
    scatter = _make_scatter(N, Ec, vdim, chunk)

    sps = []
    for i in range(K):
        sl = slice(i * Ec, (i + 1) * Ec)
        G0, G1 = gather(tab, idx0[sl], idx1[sl])
        V = _run_edge_mlp(G0, G1, edge_attr[sl], wa, wb, wea, wd, be1p,
                          we2p, be2r, Wc1, bc1r, wc2r, bc2r, pdim=pdim,
                          cdim=cdim, eb=640)
        Sp = scatter(idx1[sl], V, z)
        sps.append(Sp[0])
        sps.append(Sp[1])

    co, hidden = _run_node_mlp(feats, ctab, sps,
                               wn1f, wn1m, bn1r, Wn2, bn2r, mdim=mdim,
                               nb=2000)

    return jnp.concatenate([co[:, :pos_dim], hidden], axis=-1)
